# baseline (speedup 1.0000x reference)
import functools

import jax
import jax.numpy as jnp
from jax import lax
from jax.experimental import pallas as pl
from jax.experimental.pallas import tpu as pltpu
from jax.experimental.pallas import tpu_sc as plsc

B = 16384
D = 64
L = 16
NC, NS = 2, 16
NW = NC * NS
BPW = B // NW
GCH = 128
NCHUNK = BPW // GCH
NROWS = 1000000
TBLKC = 16384
TSTEPS = -(-NROWS // TBLKC)


def _tc_scan_body(u_ref, i_ref, w2_ref, su_ref, si_ref):
    w2 = w2_ref[...]
    ru = lax.dot_general(w2, u_ref[...], (((1,), (0,)), ((), ())),
                         preferred_element_type=jnp.float32)
    ri = lax.dot_general(w2, i_ref[...], (((1,), (0,)), ((), ())),
                         preferred_element_type=jnp.float32)
    su_ref[...] = ru[0]
    si_ref[...] = ri[1]


def _tc_scan(ut, it, w2):
    return pl.pallas_call(
        _tc_scan_body,
        grid=(TSTEPS,),
        in_specs=[
            pl.BlockSpec((D, TBLKC), lambda i: (0, i)),
            pl.BlockSpec((D, TBLKC), lambda i: (0, i)),
            pl.BlockSpec((2, D), lambda i: (0, 0)),
        ],
        out_specs=[
            pl.BlockSpec((TBLKC,), lambda i: (i,)),
            pl.BlockSpec((TBLKC,), lambda i: (i,)),
        ],
        out_shape=[
            jax.ShapeDtypeStruct((NROWS,), jnp.float32),
            jax.ShapeDtypeStruct((NROWS,), jnp.float32),
        ],
    )(ut, it, w2)


def _sc_gather_impl(su_hbm, si_hbm, tx_hbm, bias_hbm, out_hbm,
                    uidx_v, iidx_v, ubuf, ibuf, outv, bv, sem):
    wid = lax.axis_index("s") * NC + lax.axis_index("c")
    base = wid * BPW

    for j in range(NCHUNK):
        pltpu.sync_copy(tx_hbm.at[0, pl.ds(base + j * GCH, GCH)],
                        uidx_v.at[j])
        pltpu.sync_copy(tx_hbm.at[1, pl.ds(base + j * GCH, GCH)],
                        iidx_v.at[j])
    pltpu.sync_copy(bias_hbm, bv)

    copies = []
    for j in range(NCHUNK):
        copies.append(pltpu.async_copy(
            su_hbm.at[uidx_v.at[j]], ubuf.at[pl.ds(j * GCH, GCH)], sem))
        copies.append(pltpu.async_copy(
            si_hbm.at[iidx_v.at[j]], ibuf.at[pl.ds(j * GCH, GCH)], sem))
    for c in copies:
        c.wait()

    bias = bv[...]
    for k in range(BPW // L):
        outv[pl.ds(k * L, L)] = (
            ubuf[pl.ds(k * L, L)] + ibuf[pl.ds(k * L, L)] + bias)

    pltpu.sync_copy(outv, out_hbm.at[pl.ds(base, BPW)])


@functools.cache
def _build_sc_gather():
    mesh = plsc.VectorSubcoreMesh(
        core_axis_name="c", subcore_axis_name="s",
        num_cores=NC, num_subcores=NS,
    )
    return pl.kernel(
        _sc_gather_impl,
        out_type=jax.ShapeDtypeStruct((B,), jnp.float32),
        mesh=mesh,
        scratch_types=[
            pltpu.VMEM((NCHUNK, GCH), jnp.int32),
            pltpu.VMEM((NCHUNK, GCH), jnp.int32),
            pltpu.VMEM((BPW,), jnp.float32),
            pltpu.VMEM((BPW,), jnp.float32),
            pltpu.VMEM((BPW,), jnp.float32),
            pltpu.VMEM((L,), jnp.float32),
            pltpu.SemaphoreType.DMA,
        ],
        compiler_params=pltpu.CompilerParams(use_tc_tiling_on_sc=False),
    )


def kernel(train_x, u_emb, i_emb, W, b):
    w2 = W.reshape(2, D).astype(jnp.float32)
    bias16 = jnp.full((L,), b.reshape(-1)[0], jnp.float32)
    su, si = _tc_scan(u_emb.T, i_emb.T, w2)
    return _build_sc_gather()(su, si, train_x.T, bias16)
